# baseline (speedup 1.0000x reference)
import jax
import jax.numpy as jnp
from jax import lax
from jax.experimental import pallas as pl

N_BINS = 1024
INPUT_DIM = 64
BN = 2048
LANES = 128


def _vq_kernel(x_ref, x2_ref, v_ref, v2_ref, out_ref):
    xb = x_ref[...] * (-2.0)
    ab2 = lax.dot_general(
        xb, v_ref[...], (((1,), (1,)), ((), ())),
        preferred_element_type=jnp.float32,
    )
    x2 = x2_ref[...]
    mbits = None
    bc = None
    for c in range(N_BINS // LANES):
        sl = slice(c * LANES, (c + 1) * LANES)
        s = x2 + v2_ref[:, sl]
        d2 = s + ab2[:, sl]
        dist = jnp.sqrt(jnp.maximum(d2, 0.0))
        b = lax.bitcast_convert_type(dist, jnp.int32)
        if c == 0:
            mbits = b
            bc = jnp.zeros_like(b)
        else:
            lt = b < mbits
            mbits = jnp.where(lt, b, mbits)
            bc = jnp.where(lt, c, bc)
    lane = lax.broadcasted_iota(jnp.int32, mbits.shape, 1)
    idx128 = bc * LANES + lane
    mv = jnp.min(mbits, axis=1, keepdims=True)
    idx = jnp.min(jnp.where(mbits == mv, idx128, N_BINS), axis=1, keepdims=True)
    out_ref[...] = idx


def kernel(x, vectors):
    shape = x.shape[:-1]
    flat = x.reshape(-1, x.shape[-1])
    n = flat.shape[0]
    x2 = jnp.sum(flat * flat, axis=-1, keepdims=True)
    v2 = jnp.sum(vectors * vectors, axis=-1)[None, :]

    grid = (n // BN,)
    out = pl.pallas_call(
        _vq_kernel,
        grid=grid,
        in_specs=[
            pl.BlockSpec((BN, INPUT_DIM), lambda i: (i, 0)),
            pl.BlockSpec((BN, 1), lambda i: (i, 0)),
            pl.BlockSpec((N_BINS, INPUT_DIM), lambda i: (0, 0)),
            pl.BlockSpec((1, N_BINS), lambda i: (0, 0)),
        ],
        out_specs=pl.BlockSpec((BN, 1), lambda i: (i, 0)),
        out_shape=jax.ShapeDtypeStruct((n, 1), jnp.int32),
    )(flat, x2, vectors, v2)
    return out.reshape(shape)
